# baseline (speedup 1.0000x reference)
import functools

import jax
import jax.numpy as jnp
from jax import lax
from jax.experimental import pallas as pl
from jax.experimental.pallas import tpu as pltpu
from jax.experimental.pallas import tpu_sc as plsc

CB = 8192
D = 256
BM = 512
CHUNKS = (2736, 2736, 2720)
COMMIT = 0.25

NC = 2
NS = 16
NW = NC * NS
GCHUNK = 128

def _dist_argmin_body(x_ref, e_ref, idx_ref):
    x = x_ref[...]
    xb = x.astype(jnp.bfloat16)
    x_norm = jnp.sum(x * x, axis=1, keepdims=True)

    gv = jnp.full((BM, 1), jnp.inf, dtype=jnp.float32)
    gi = jnp.full((BM, 1), 2**31 - 1, dtype=jnp.int32)
    off = 0
    for csize in CHUNKS:
        e = e_ref[pl.ds(off, csize), :]
        eb = e.astype(jnp.bfloat16)
        dot = lax.dot_general(xb, eb, (((1,), (1,)), ((), ())),
                              preferred_element_type=jnp.float32)
        e_norm = jnp.sum(e * e, axis=1)[None, :]
        dist = (x_norm + e_norm) - 2.0 * dot
        cv = jnp.min(dist, axis=1, keepdims=True)
        ids = lax.broadcasted_iota(jnp.int32, dist.shape, 1) + off
        ci = jnp.min(jnp.where(dist == cv, ids, jnp.int32(2**30)),
                     axis=1, keepdims=True)
        take = (cv < gv) | ((cv == gv) & (ci < gi))
        gv = jnp.where(take, cv.astype(jnp.bfloat16).astype(jnp.float32), gv)
        gi = jnp.where(take, ci, gi)
        off += csize
    idx_ref[...] = gi


def _loss_body(x_ref, q_ref, loss_ref):
    diff = q_ref[...] - x_ref[...]
    loss_ref[...] = (1.0 + COMMIT) * jnp.sum(diff * diff, axis=1,
                                             keepdims=True)


def _gather_body(nch, table, idxs, out, idx_v, rows_v, sem):
    wid = lax.axis_index("s") * NC + lax.axis_index("c")
    pltpu.sync_copy(idxs.at[wid], idx_v)
    for c in range(nch):
        pltpu.async_copy(table.at[idx_v.at[c]], rows_v, sem).wait()
        pltpu.sync_copy(rows_v, out.at[pl.ds((wid * nch + c) * GCHUNK, GCHUNK)])


def _gather_rows(embedding, idx):
    m = idx.shape[0]
    nch = (m // NW) // GCHUNK
    idx3 = idx.reshape(NW, nch, GCHUNK)
    mesh = plsc.VectorSubcoreMesh(core_axis_name="c", subcore_axis_name="s")
    return pl.kernel(
        functools.partial(_gather_body, nch),
        out_type=jax.ShapeDtypeStruct((m, D), jnp.float32),
        mesh=mesh,
        scratch_types=[
            pltpu.VMEM((nch, GCHUNK), jnp.int32),
            pltpu.VMEM((GCHUNK, D), jnp.float32),
            pltpu.SemaphoreType.DMA,
        ],
    )(embedding, idx3)


def kernel(inputs, embedding):
    input_shape = inputs.shape
    x = inputs.reshape(-1, D)
    m = x.shape[0]
    idx2 = pl.pallas_call(
        _dist_argmin_body,
        grid=(m // BM,),
        in_specs=[
            pl.BlockSpec((BM, D), lambda i: (i, 0)),
            pl.BlockSpec((CB, D), lambda i: (0, 0)),
        ],
        out_specs=pl.BlockSpec((BM, 1), lambda i: (i, 0)),
        out_shape=jax.ShapeDtypeStruct((m, 1), jnp.int32),
    )(x, embedding)
    idx = idx2.reshape(m)
    quantized = _gather_rows(embedding, idx)
    loss2 = pl.pallas_call(
        _loss_body,
        grid=(m // BM,),
        in_specs=[
            pl.BlockSpec((BM, D), lambda i: (i, 0)),
            pl.BlockSpec((BM, D), lambda i: (i, 0)),
        ],
        out_specs=pl.BlockSpec((BM, 1), lambda i: (i, 0)),
        out_shape=jax.ShapeDtypeStruct((m, 1), jnp.float32),
    )(x, quantized)
    return (quantized.reshape(input_shape),
            idx.reshape(input_shape[:-1]),
            loss2.reshape(input_shape[:-1]))
